# baseline (speedup 1.0000x reference)
import functools

import jax
import jax.numpy as jnp
from jax.experimental import pallas as pl


def _matmul_small_body(x_ref, w_ref, out_ref):
    out_ref[...] = jnp.dot(
        x_ref[...], w_ref[...], preferred_element_type=jnp.float32
    )


def _layer1_body(adj_ref, s1_ref, b1_ref, scale_ref, shift_ref, w2_ref, out_ref):
    h = jnp.dot(adj_ref[...], s1_ref[...], preferred_element_type=jnp.float32)
    h = h + b1_ref[...]
    h = jnp.where(h >= 0, h, 0.01 * h)
    h = h * scale_ref[...] + shift_ref[...]
    out_ref[...] = jnp.dot(h, w2_ref[...], preferred_element_type=jnp.float32)


def _layer2_body(adj_ref, s2_ref, b2_ref, out_ref):
    out_ref[...] = (
        jnp.dot(adj_ref[...], s2_ref[...], preferred_element_type=jnp.float32)
        + b2_ref[...]
    )


@functools.partial(jax.jit, static_argnames=("bm",))
def _gcn_forward(x, adj, W1, b1, scale, shift, W2, b2, bm):
    n, f_in = x.shape
    h_dim = W1.shape[1]
    c_dim = W2.shape[1]

    b1r = b1.reshape(1, h_dim)
    scaler = scale.reshape(1, h_dim)
    shiftr = shift.reshape(1, h_dim)
    b2r = b2.reshape(1, c_dim)

    s1 = pl.pallas_call(
        _matmul_small_body,
        grid=(n // bm,),
        in_specs=[
            pl.BlockSpec((bm, f_in), lambda i: (i, 0)),
            pl.BlockSpec((f_in, h_dim), lambda i: (0, 0)),
        ],
        out_specs=pl.BlockSpec((bm, h_dim), lambda i: (i, 0)),
        out_shape=jax.ShapeDtypeStruct((n, h_dim), jnp.float32),
    )(x, W1)

    s2 = pl.pallas_call(
        _layer1_body,
        grid=(n // bm,),
        in_specs=[
            pl.BlockSpec((bm, n), lambda i: (i, 0)),
            pl.BlockSpec((n, h_dim), lambda i: (0, 0)),
            pl.BlockSpec((1, h_dim), lambda i: (0, 0)),
            pl.BlockSpec((1, h_dim), lambda i: (0, 0)),
            pl.BlockSpec((1, h_dim), lambda i: (0, 0)),
            pl.BlockSpec((h_dim, c_dim), lambda i: (0, 0)),
        ],
        out_specs=pl.BlockSpec((bm, c_dim), lambda i: (i, 0)),
        out_shape=jax.ShapeDtypeStruct((n, c_dim), jnp.float32),
    )(adj, s1, b1r, scaler, shiftr, W2)

    out = pl.pallas_call(
        _layer2_body,
        grid=(n // bm,),
        in_specs=[
            pl.BlockSpec((bm, n), lambda i: (i, 0)),
            pl.BlockSpec((n, c_dim), lambda i: (0, 0)),
            pl.BlockSpec((1, c_dim), lambda i: (0, 0)),
        ],
        out_specs=pl.BlockSpec((bm, c_dim), lambda i: (i, 0)),
        out_shape=jax.ShapeDtypeStruct((n, c_dim), jnp.float32),
    )(adj, s2, b2r)
    return out


def kernel(x, adj, W1, b1, gamma, beta, running_mean, running_var, W2, b2):
    scale = gamma * jax.lax.rsqrt(running_var + 1e-5)
    shift = beta - running_mean * scale
    n = x.shape[0]
    bm = 200 if n % 200 == 0 else n
    return _gcn_forward(x, adj, W1, b1, scale, shift, W2, b2, bm)
